# baseline (speedup 1.0000x reference)
import functools

import jax
import jax.numpy as jnp
from jax import lax
from jax.experimental import pallas as pl
from jax.experimental.pallas import tpu as pltpu
from jax.experimental.pallas import tpu_sc as plsc

B, S, D = 8, 512, 768
NC, NS, L = 2, 16, 16
NW = NC * NS
DB = D // L
GROUPS = B * DB
GPW = GROUPS // NW
WPB = NW // B


def _exp_poly(x):
    e = 1.0 / 5040.0 + x * (1.0 / 40320.0)
    e = 1.0 / 720.0 + x * e
    e = 1.0 / 120.0 + x * e
    e = 1.0 / 24.0 + x * e
    e = 1.0 / 6.0 + x * e
    e = 0.5 + x * e
    e = 1.0 + x * e
    return 1.0 + x * e


def _heat_body(h_hbm, t_hbm, p_hbm, out_hbm, p_v, t_v, r_v, hb_v):
    wid = lax.axis_index("s") * NC + lax.axis_index("c")
    b = wid // WPB

    pltpu.sync_copy(p_hbm, p_v)
    pltpu.sync_copy(t_hbm.at[b], t_v)

    zeros_i = jnp.zeros((L,), jnp.int32)
    iota = lax.iota(jnp.int32, L)
    eps_v = plsc.load_gather(p_v, [zeros_i + 1])
    beta_v = plsc.load_gather(p_v, [zeros_i + 2])

    def build(c, _):
        idx = c * L + iota
        prev = jnp.where(idx > 0, idx - 1, 0)
        dt = t_v[pl.ds(c * L, L)] - plsc.load_gather(t_v, [prev])
        r_v[pl.ds(c * L, L)] = _exp_poly(beta_v * dt)
        return 0
    lax.fori_loop(0, S // L, build, 0)

    zf = jnp.zeros((L,), jnp.float32)
    for i in range(GPW):
        d0 = ((wid * GPW + i) % DB) * L
        pltpu.sync_copy(h_hbm.at[b, :, pl.ds(d0, L)], hb_v)

        def step(s, carry):
            a1, a2 = carry
            hv = hb_v[s]
            rv = plsc.load_gather(r_v, [jnp.full((L,), s, jnp.int32)])
            a1 = a1 + hv
            a2 = rv * a2 + jnp.maximum(hv, 0.0)
            hb_v[s] = a1 + eps_v * a2
            return (a1, a2)
        lax.fori_loop(0, S, step, (zf, zf))

        pltpu.sync_copy(hb_v, out_hbm.at[b, :, pl.ds(d0, L)])


@jax.jit
def _heat(h, t, params):
    mesh = plsc.VectorSubcoreMesh(core_axis_name="c", subcore_axis_name="s")
    f = functools.partial(
        pl.kernel,
        out_type=jax.ShapeDtypeStruct((B, S, D), jnp.float32),
        mesh=mesh,
        scratch_types=[
            pltpu.VMEM((L,), jnp.float32),
            pltpu.VMEM((S,), jnp.float32),
            pltpu.VMEM((S,), jnp.float32),
            pltpu.VMEM((S, L), jnp.float32),
        ],
        compiler_params=pltpu.CompilerParams(
            use_tc_tiling_on_sc=False, needs_layout_passes=False),
    )(_heat_body)
    return f(h, t, params)


def kernel(h, t, epsilon, beta):
    params = jnp.zeros((L,), jnp.float32)
    params = params.at[1].set(epsilon).at[2].set(beta)
    return _heat(h.astype(jnp.float32), t.astype(jnp.float32), params)


# speedup vs baseline: 2.0108x; 2.0108x over previous
import functools

import jax
import jax.numpy as jnp
from jax import lax
from jax.experimental import pallas as pl
from jax.experimental.pallas import tpu as pltpu
from jax.experimental.pallas import tpu_sc as plsc

B, S, D = 8, 512, 768
NC, NS, L = 2, 16, 16
NW = NC * NS
PW = 128
CPP = PW // L
NP = B * (D // PW)
DPB = D // PW
SH = S // 2
UNROLL = 4


def _exp_poly(x):
    e = 1.0 / 5040.0 + x * (1.0 / 40320.0)
    e = 1.0 / 720.0 + x * e
    e = 1.0 / 120.0 + x * e
    e = 1.0 / 24.0 + x * e
    e = 1.0 / 6.0 + x * e
    e = 0.5 + x * e
    e = 1.0 + x * e
    return 1.0 + x * e


def _heat_body(h_hbm, t_hbm, p_hbm, out_hbm,
               p_v, t_v, r_v, hb_v, sin0, sin1, sin2, sout0, sout1, sout2):
    wid = lax.axis_index("s") * NC + lax.axis_index("c")
    heavy = wid < NP - NW
    sins = [sin0, sin1, sin2]
    souts = [sout0, sout1, sout2]

    pltpu.sync_copy(p_hbm, p_v)

    zeros_i = jnp.zeros((L,), jnp.int32)
    iota = lax.iota(jnp.int32, L)
    eps_v = plsc.load_gather(p_v, [zeros_i + 1])
    beta_v = plsc.load_gather(p_v, [zeros_i + 2])

    def pb(p):
        return p // DPB, (p % DPB) * PW

    def src(p, k, buf):
        b, d0 = pb(p)
        return h_hbm.at[b, pl.ds(k * SH, SH), pl.ds(d0, PW)], hb_v.at[buf]

    def dst(p, k, buf):
        b, d0 = pb(p)
        return hb_v.at[buf], out_hbm.at[b, pl.ds(k * SH, SH), pl.ds(d0, PW)]

    def build_r(p):
        b, _ = pb(p)
        pltpu.sync_copy(t_hbm.at[b], t_v)

        def build(c, _):
            idx = c * L + iota
            prev = jnp.where(idx > 0, idx - 1, 0)
            dt = t_v[pl.ds(c * L, L)] - plsc.load_gather(t_v, [prev])
            r_v[pl.ds(c * L, L)] = _exp_poly(beta_v * dt)
            return 0
        lax.fori_loop(0, S // L, build, 0)

    def scan_half(buf, k, carry):
        hb = hb_v.at[buf]

        def step(it, c):
            for u in range(UNROLL):
                s = it * UNROLL + u
                rv = plsc.load_gather(r_v, [jnp.full((L,), k * SH + s,
                                                     jnp.int32)])
                for ci in range(CPP):
                    a1, a2 = c[2 * ci], c[2 * ci + 1]
                    hv = hb[s, pl.ds(ci * L, L)]
                    a1 = a1 + hv
                    a2 = rv * a2 + eps_v * jnp.maximum(hv, 0.0)
                    hb[s, pl.ds(ci * L, L)] = a1 + a2
                    c = c[:2 * ci] + (a1, a2) + c[2 * ci + 2:]
            return c
        return lax.fori_loop(0, SH // UNROLL, step, carry)

    zf = jnp.zeros((L,), jnp.float32)
    zero_carry = (zf,) * (2 * CPP)

    p0, p1 = wid, wid + NW
    in_dma_args = [src(p0, 0, 0), src(p0, 1, 1), src(p1, 0, 2), src(p1, 1, 0)]
    out_dma_args = [dst(p0, 0, 0), dst(p0, 1, 1), dst(p1, 0, 2), dst(p1, 1, 0)]
    bufs = [0, 1, 2, 0]

    pltpu.async_copy(*in_dma_args[0], sins[0])
    pltpu.async_copy(*in_dma_args[1], sins[1])
    build_r(p0)

    pltpu.make_async_copy(*in_dma_args[0], sins[0]).wait()

    @pl.when(heavy)
    def _():
        pltpu.async_copy(*in_dma_args[2], sins[2])
    carry = scan_half(0, 0, zero_carry)
    pltpu.async_copy(*out_dma_args[0], souts[0])

    pltpu.make_async_copy(*in_dma_args[1], sins[1]).wait()
    scan_half(1, 1, carry)
    pltpu.async_copy(*out_dma_args[1], souts[1])

    @pl.when(heavy)
    def _():
        build_r(p1)
        pltpu.make_async_copy(*in_dma_args[2], sins[2]).wait()
        pltpu.make_async_copy(*out_dma_args[0], souts[0]).wait()
        pltpu.async_copy(*in_dma_args[3], sins[0])
        carry2 = scan_half(2, 0, zero_carry)
        pltpu.async_copy(*out_dma_args[2], souts[2])

        pltpu.make_async_copy(*in_dma_args[3], sins[0]).wait()
        scan_half(0, 1, carry2)
        pltpu.async_copy(*out_dma_args[3], souts[0])

        pltpu.make_async_copy(*out_dma_args[2], souts[2]).wait()
        pltpu.make_async_copy(*out_dma_args[3], souts[0]).wait()

    pltpu.make_async_copy(*out_dma_args[1], souts[1]).wait()

    @pl.when(jnp.logical_not(heavy))
    def _():
        pltpu.make_async_copy(*out_dma_args[0], souts[0]).wait()


@jax.jit
def _heat(h, t, params):
    mesh = plsc.VectorSubcoreMesh(core_axis_name="c", subcore_axis_name="s")
    f = functools.partial(
        pl.kernel,
        out_type=jax.ShapeDtypeStruct((B, S, D), jnp.float32),
        mesh=mesh,
        scratch_types=[
            pltpu.VMEM((L,), jnp.float32),
            pltpu.VMEM((S,), jnp.float32),
            pltpu.VMEM((S,), jnp.float32),
            pltpu.VMEM((3, SH, PW), jnp.float32),
            pltpu.SemaphoreType.DMA,
            pltpu.SemaphoreType.DMA,
            pltpu.SemaphoreType.DMA,
            pltpu.SemaphoreType.DMA,
            pltpu.SemaphoreType.DMA,
            pltpu.SemaphoreType.DMA,
        ],
        compiler_params=pltpu.CompilerParams(
            use_tc_tiling_on_sc=True, needs_layout_passes=False),
    )(_heat_body)
    return f(h, t, params)


def kernel(h, t, epsilon, beta):
    params = jnp.zeros((L,), jnp.float32)
    params = params.at[1].set(epsilon).at[2].set(beta)
    return _heat(h.astype(jnp.float32), t.astype(jnp.float32), params)


# speedup vs baseline: 3.4619x; 1.7217x over previous
import functools

import jax
import jax.numpy as jnp
from jax import lax
from jax.experimental import pallas as pl
from jax.experimental.pallas import tpu as pltpu
from jax.experimental.pallas import tpu_sc as plsc

B, S, D = 8, 512, 768
NC, NS, L = 2, 16, 16
NW = NC * NS
PW = 128
CPP = PW // L
DSC = 512
DTC = D - DSC
PPB = DSC // PW
SH = S // 2
UNROLL = 4


def _exp_poly(x):
    e = 1.0 / 5040.0 + x * (1.0 / 40320.0)
    e = 1.0 / 720.0 + x * e
    e = 1.0 / 120.0 + x * e
    e = 1.0 / 24.0 + x * e
    e = 1.0 / 6.0 + x * e
    e = 0.5 + x * e
    e = 1.0 + x * e
    return 1.0 + x * e


def _sc_body(h_hbm, t_hbm, p_hbm, out_hbm,
             p_v, t_v, r_v, hb_v, sin0, sin1, sout0, sout1):
    wid = lax.axis_index("s") * NC + lax.axis_index("c")
    b = wid // PPB
    d0 = (wid % PPB) * PW

    srcs = [h_hbm.at[b, pl.ds(k * SH, SH), pl.ds(d0, PW)] for k in (0, 1)]
    dsts = [out_hbm.at[b, pl.ds(k * SH, SH), pl.ds(d0, PW)] for k in (0, 1)]
    sins = [sin0, sin1]
    souts = [sout0, sout1]

    for k in (0, 1):
        pltpu.async_copy(srcs[k], hb_v.at[k], sins[k])

    pltpu.sync_copy(p_hbm, p_v)
    pltpu.sync_copy(t_hbm.at[b], t_v)

    zeros_i = jnp.zeros((L,), jnp.int32)
    iota = lax.iota(jnp.int32, L)
    eps_v = plsc.load_gather(p_v, [zeros_i + 1])
    beta_v = plsc.load_gather(p_v, [zeros_i + 2])

    def build(c, _):
        idx = c * L + iota
        prev = jnp.where(idx > 0, idx - 1, 0)
        dt = t_v[pl.ds(c * L, L)] - plsc.load_gather(t_v, [prev])
        r_v[pl.ds(c * L, L)] = _exp_poly(beta_v * dt)
        return 0
    lax.fori_loop(0, S // L, build, 0)

    def scan_half(k, carry):
        hb = hb_v.at[k]

        def step(it, c):
            for u in range(UNROLL):
                s = it * UNROLL + u
                rv = plsc.load_gather(
                    r_v, [jnp.full((L,), k * SH + s, jnp.int32)])
                hvs = [hb[s, pl.ds(ci * L, L)] for ci in range(CPP)]
                a1s = [c[2 * ci] + hvs[ci] for ci in range(CPP)]
                rps = [jnp.maximum(hvs[ci], 0.0) for ci in range(CPP)]
                eps_rps = [eps_v * rps[ci] for ci in range(CPP)]
                ra2s = [rv * c[2 * ci + 1] for ci in range(CPP)]
                a2s = [ra2s[ci] + eps_rps[ci] for ci in range(CPP)]
                for ci in range(CPP):
                    hb[s, pl.ds(ci * L, L)] = a1s[ci] + a2s[ci]
                c = tuple(x for ci in range(CPP) for x in (a1s[ci], a2s[ci]))
            return c
        return lax.fori_loop(0, SH // UNROLL, step, carry)

    zf = jnp.zeros((L,), jnp.float32)
    pltpu.make_async_copy(srcs[0], hb_v.at[0], sins[0]).wait()
    carry = scan_half(0, (zf,) * (2 * CPP))
    pltpu.async_copy(hb_v.at[0], dsts[0], souts[0])

    pltpu.make_async_copy(srcs[1], hb_v.at[1], sins[1]).wait()
    scan_half(1, carry)
    pltpu.async_copy(hb_v.at[1], dsts[1], souts[1])

    pltpu.make_async_copy(hb_v.at[0], dsts[0], souts[0]).wait()
    pltpu.make_async_copy(hb_v.at[1], dsts[1], souts[1]).wait()


def _tc_body(trow_ref, tcol_ref, p_ref, h_ref, out_ref):
    eps = p_ref[0, 0]
    beta = p_ref[0, 1]
    tau = tcol_ref[0] - trow_ref[0]
    mask = (tau >= 0.0).astype(jnp.float32)
    decay = (eps * mask) * jnp.exp(beta * tau)
    hb = h_ref[0]
    out_ref[0] = (
        jnp.dot(mask, hb, preferred_element_type=jnp.float32)
        + jnp.dot(decay, jnp.maximum(hb, 0.0),
                  preferred_element_type=jnp.float32))


@jax.jit
def _heat(h, t, params):
    mesh = plsc.VectorSubcoreMesh(core_axis_name="c", subcore_axis_name="s")
    sc = functools.partial(
        pl.kernel,
        out_type=jax.ShapeDtypeStruct((B, S, D), jnp.float32),
        mesh=mesh,
        scratch_types=[
            pltpu.VMEM((L,), jnp.float32),
            pltpu.VMEM((S,), jnp.float32),
            pltpu.VMEM((S,), jnp.float32),
            pltpu.VMEM((2, SH, PW), jnp.float32),
            pltpu.SemaphoreType.DMA,
            pltpu.SemaphoreType.DMA,
            pltpu.SemaphoreType.DMA,
            pltpu.SemaphoreType.DMA,
        ],
        compiler_params=pltpu.CompilerParams(
            use_tc_tiling_on_sc=True, needs_layout_passes=False),
    )(_sc_body)
    sc_full = sc(h, t, params)

    tc_out = pl.pallas_call(
        _tc_body,
        grid=(B,),
        in_specs=[
            pl.BlockSpec((1, 1, S), lambda b: (b, 0, 0)),
            pl.BlockSpec((1, S, 1), lambda b: (b, 0, 0)),
            pl.BlockSpec(memory_space=pltpu.SMEM),
            pl.BlockSpec((1, S, DTC), lambda b: (b, 0, DSC // DTC)),
        ],
        out_specs=pl.BlockSpec((1, S, DTC), lambda b: (b, 0, 0)),
        out_shape=jax.ShapeDtypeStruct((B, S, DTC), jnp.float32),
    )(t[:, None, :], t[:, :, None], params[None, 1:3], h)

    return lax.dynamic_update_slice(sc_full, tc_out, (0, 0, DSC))


def kernel(h, t, epsilon, beta):
    params = jnp.zeros((L,), jnp.float32)
    params = params.at[1].set(epsilon).at[2].set(beta)
    return _heat(h.astype(jnp.float32), t.astype(jnp.float32), params)


# speedup vs baseline: 3.4721x; 1.0030x over previous
import functools

import jax
import jax.numpy as jnp
from jax import lax
from jax.experimental import pallas as pl
from jax.experimental.pallas import tpu as pltpu
from jax.experimental.pallas import tpu_sc as plsc

B, S, D = 8, 512, 768
NC, NS, L = 2, 16, 16
NW = NC * NS
PW = 128
CPP = PW // L
DSC = 512
DTC = D - DSC
PPB = DSC // PW
SH = S // 2
UNROLL = 4


def _exp_poly(x):
    e = 1.0 / 5040.0 + x * (1.0 / 40320.0)
    e = 1.0 / 720.0 + x * e
    e = 1.0 / 120.0 + x * e
    e = 1.0 / 24.0 + x * e
    e = 1.0 / 6.0 + x * e
    e = 0.5 + x * e
    e = 1.0 + x * e
    return 1.0 + x * e


def _sc_body(h_hbm, t_hbm, p_hbm, out_hbm,
             p_v, t_v, r_v, hb_v, sin0, sin1, sout0, sout1):
    wid = lax.axis_index("s") * NC + lax.axis_index("c")
    b = wid // PPB
    d0 = (wid % PPB) * PW

    srcs = [h_hbm.at[b, pl.ds(k * SH, SH), pl.ds(d0, PW)] for k in (0, 1)]
    dsts = [out_hbm.at[b, pl.ds(k * SH, SH), pl.ds(d0, PW)] for k in (0, 1)]
    sins = [sin0, sin1]
    souts = [sout0, sout1]

    for k in (0, 1):
        pltpu.async_copy(srcs[k], hb_v.at[k], sins[k])

    pltpu.sync_copy(p_hbm, p_v)
    pltpu.sync_copy(t_hbm.at[b], t_v)

    zeros_i = jnp.zeros((L,), jnp.int32)
    iota = lax.iota(jnp.int32, L)
    eps_v = plsc.load_gather(p_v, [zeros_i + 1])
    beta_v = plsc.load_gather(p_v, [zeros_i + 2])

    def build(c, _):
        idx = c * L + iota
        prev = jnp.where(idx > 0, idx - 1, 0)
        dt = t_v[pl.ds(c * L, L)] - plsc.load_gather(t_v, [prev])
        r_v[pl.ds(c * L, L)] = _exp_poly(beta_v * dt)
        return 0
    lax.fori_loop(0, S // L, build, 0)

    zf = jnp.zeros((L,), jnp.float32)

    def half(k, carry):
        hb = hb_v.at[k]

        @pl.when(k == 0)
        def _():
            pltpu.make_async_copy(srcs[0], hb_v.at[0], sins[0]).wait()

        @pl.when(k == 1)
        def _():
            pltpu.make_async_copy(srcs[1], hb_v.at[1], sins[1]).wait()

        def step(it, c):
            for u in range(UNROLL):
                s = it * UNROLL + u
                rv = plsc.load_gather(
                    r_v, [jnp.full((L,), k * SH + s, jnp.int32)])
                hvs = [hb[s, pl.ds(ci * L, L)] for ci in range(CPP)]
                a1s = [c[2 * ci] + hvs[ci] for ci in range(CPP)]
                rps = [jnp.maximum(hvs[ci], 0.0) for ci in range(CPP)]
                eps_rps = [eps_v * rps[ci] for ci in range(CPP)]
                ra2s = [rv * c[2 * ci + 1] for ci in range(CPP)]
                a2s = [ra2s[ci] + eps_rps[ci] for ci in range(CPP)]
                for ci in range(CPP):
                    hb[s, pl.ds(ci * L, L)] = a1s[ci] + a2s[ci]
                c = tuple(x for ci in range(CPP) for x in (a1s[ci], a2s[ci]))
            return c
        carry = lax.fori_loop(0, SH // UNROLL, step, carry)

        @pl.when(k == 0)
        def _():
            pltpu.async_copy(hb_v.at[0], dsts[0], souts[0])

        @pl.when(k == 1)
        def _():
            pltpu.async_copy(hb_v.at[1], dsts[1], souts[1])
        return carry

    lax.fori_loop(0, 2, half, (zf,) * (2 * CPP))

    pltpu.make_async_copy(hb_v.at[0], dsts[0], souts[0]).wait()
    pltpu.make_async_copy(hb_v.at[1], dsts[1], souts[1]).wait()


def _tc_body(trow_ref, tcol_ref, p_ref, h_ref, out_ref):
    eps = p_ref[0, 0]
    beta = p_ref[0, 1]
    tau = tcol_ref[0] - trow_ref[0]
    mask = (tau >= 0.0).astype(jnp.float32)
    decay = (eps * mask) * jnp.exp(beta * tau)
    hb = h_ref[0]
    out_ref[0] = (
        jnp.dot(mask, hb, preferred_element_type=jnp.float32)
        + jnp.dot(decay, jnp.maximum(hb, 0.0),
                  preferred_element_type=jnp.float32))


@jax.jit
def _heat(h, t, params):
    mesh = plsc.VectorSubcoreMesh(core_axis_name="c", subcore_axis_name="s")
    sc = functools.partial(
        pl.kernel,
        out_type=jax.ShapeDtypeStruct((B, S, D), jnp.float32),
        mesh=mesh,
        scratch_types=[
            pltpu.VMEM((L,), jnp.float32),
            pltpu.VMEM((S,), jnp.float32),
            pltpu.VMEM((S,), jnp.float32),
            pltpu.VMEM((2, SH, PW), jnp.float32),
            pltpu.SemaphoreType.DMA,
            pltpu.SemaphoreType.DMA,
            pltpu.SemaphoreType.DMA,
            pltpu.SemaphoreType.DMA,
        ],
        compiler_params=pltpu.CompilerParams(
            use_tc_tiling_on_sc=True, needs_layout_passes=False,
            skip_device_barrier=True),
    )(_sc_body)
    sc_full = sc(h, t, params)

    tc_out = pl.pallas_call(
        _tc_body,
        grid=(B,),
        in_specs=[
            pl.BlockSpec((1, 1, S), lambda b: (b, 0, 0)),
            pl.BlockSpec((1, S, 1), lambda b: (b, 0, 0)),
            pl.BlockSpec(memory_space=pltpu.SMEM),
            pl.BlockSpec((1, S, DTC), lambda b: (b, 0, DSC // DTC)),
        ],
        out_specs=pl.BlockSpec((1, S, DTC), lambda b: (b, 0, 0)),
        out_shape=jax.ShapeDtypeStruct((B, S, DTC), jnp.float32),
    )(t[:, None, :], t[:, :, None], params[None, 1:3], h)

    return lax.dynamic_update_slice(sc_full, tc_out, (0, 0, DSC))


def kernel(h, t, epsilon, beta):
    params = jnp.zeros((L,), jnp.float32)
    params = params.at[1].set(epsilon).at[2].set(beta)
    return _heat(h.astype(jnp.float32), t.astype(jnp.float32), params)
